# baseline (device time: 193411 ns/iter reference)
import jax
import jax.numpy as jnp
from jax import lax
from jax.experimental import pallas as pl
from jax.experimental.pallas import tpu as pltpu

N_DEV = 8
M_PER = 512
HALF = M_PER // 2
N_HOP = N_DEV - 1


def kernel(x, w_mat):
    m_per, k = x.shape
    _, n_per = w_mat.shape

    x = x.astype(jnp.bfloat16)

    def body(x_ref, w_hbm, out_hbm, cw_buf, cw_send, cw_recv, w_sem):
        me = lax.axis_index("i")
        right = lax.rem(me + 1, N_DEV)
        left = lax.rem(me + N_DEV - 1, N_DEV)

        barrier_sem = pltpu.get_barrier_semaphore()
        for nbr in (left, right):
            pl.semaphore_signal(
                barrier_sem, inc=1,
                device_id=(nbr,), device_id_type=pl.DeviceIdType.MESH,
            )
        pl.semaphore_wait(barrier_sem, 2)

        def send(src_ref, dst_ref, s_sem, r_sem, dev):
            rdma = pltpu.make_async_remote_copy(
                src_ref=src_ref, dst_ref=dst_ref,
                send_sem=s_sem, recv_sem=r_sem,
                device_id=(dev,), device_id_type=pl.DeviceIdType.MESH,
            )
            rdma.start()
            return rdma

        sends = [send(x_ref.at[pl.ds(0, HALF), :], cw_buf.at[0],
                      cw_send.at[0], cw_recv.at[0], right)]

        for h in range(N_HOP):
            pltpu.make_async_remote_copy(
                src_ref=cw_buf.at[h], dst_ref=cw_buf.at[h],
                send_sem=cw_send.at[h], recv_sem=cw_recv.at[h],
                device_id=(right,), device_id_type=pl.DeviceIdType.MESH,
            ).wait_recv()
            if h + 1 < N_HOP:
                sends.append(send(cw_buf.at[h], cw_buf.at[h + 1],
                                  cw_send.at[h + 1], cw_recv.at[h + 1],
                                  right))

        for rdma in sends:
            rdma.wait_send()

    out_shape = jax.ShapeDtypeStruct((N_DEV * M_PER, n_per), jnp.float32)
    return pl.pallas_call(
        body,
        out_shape=out_shape,
        in_specs=[
            pl.BlockSpec(memory_space=pltpu.VMEM),
            pl.BlockSpec(memory_space=pl.ANY),
        ],
        out_specs=pl.BlockSpec(memory_space=pl.ANY),
        scratch_shapes=[
            pltpu.VMEM((N_HOP, HALF, k), jnp.bfloat16),
            pltpu.SemaphoreType.DMA((N_HOP,)),
            pltpu.SemaphoreType.DMA((N_HOP,)),
            pltpu.SemaphoreType.DMA,
        ],
        compiler_params=pltpu.CompilerParams(
            collective_id=0,
            vmem_limit_bytes=64 * 1024 * 1024,
        ),
    )(x, w_mat)


# device time: 159437 ns/iter; 1.2131x vs baseline; 1.2131x over previous
import jax
import jax.numpy as jnp
from jax import lax
from jax.experimental import pallas as pl
from jax.experimental.pallas import tpu as pltpu

N_DEV = 8
M_PER = 512
N_PART = 4
ROWS = M_PER // N_PART
N_STAGE = 4

X_MASK, Y_MASK, Z_MASK = 1, 3, 4

ORDERS = (
    (Z_MASK, Y_MASK, X_MASK),
    (Y_MASK, Z_MASK, X_MASK),
    (X_MASK, Z_MASK, Y_MASK),
    (X_MASK, Y_MASK, Z_MASK),
)

PH1_ORDER = (0, 1, 2, 3)
PH2_ORDER = (0, 1, 3, 2)
PH3_ORDER = (0, 2, 3, 1)
PH_ORDERS = (PH1_ORDER, PH2_ORDER, PH3_ORDER)


def _g_table(masks):
    m1, m2, m3 = masks
    g = [0, m1, m2, m2 ^ m1]
    return g + [m3 ^ v for v in g]


def kernel(x, w_mat):
    m_per, k = x.shape
    _, n_per = w_mat.shape
    assert m_per == M_PER and k == 4096 and n_per == 1024

    x = x.astype(jnp.bfloat16)

    def body(x_ref, w_hbm, out_hbm, bufs, w_f32, w_bf, stage,
             send_sems, recv_sems, w_sem, out_sems):
        me = lax.axis_index("i")

        w_dma = pltpu.make_async_copy(w_hbm, w_f32, w_sem)
        w_dma.start()

        for p in range(N_PART):
            bufs[p, 0, :, :] = x_ref[pl.ds(p * ROWS, ROWS), :]

        barrier_sem = pltpu.get_barrier_semaphore()
        for mask in (X_MASK, Y_MASK, Z_MASK):
            pl.semaphore_signal(
                barrier_sem, inc=1,
                device_id=(me ^ mask,), device_id_type=pl.DeviceIdType.MESH,
            )
        pl.semaphore_wait(barrier_sem, 3)

        def exchange_start(p, ph):
            cnt = 1 << ph
            nbr = me ^ ORDERS[p][ph]
            rdma = pltpu.make_async_remote_copy(
                src_ref=bufs.at[p, pl.ds(0, cnt)],
                dst_ref=bufs.at[p, pl.ds(cnt, cnt)],
                send_sem=send_sems.at[p, ph],
                recv_sem=recv_sems.at[p, ph],
                device_id=(nbr,), device_id_type=pl.DeviceIdType.MESH,
            )
            rdma.start()
            return rdma

        def exchange_wait_recv(p, ph):
            cnt = 1 << ph
            nbr = me ^ ORDERS[p][ph]
            pltpu.make_async_remote_copy(
                src_ref=bufs.at[p, pl.ds(0, cnt)],
                dst_ref=bufs.at[p, pl.ds(cnt, cnt)],
                send_sem=send_sems.at[p, ph],
                recv_sem=recv_sems.at[p, ph],
                device_id=(nbr,), device_id_type=pl.DeviceIdType.MESH,
            ).wait_recv()

        g_tables = [_g_table(ORDERS[p]) for p in range(N_PART)]

        sends = [exchange_start(p, 0) for p in range(N_PART)]

        w_dma.wait()
        w_bf[:, :] = w_f32[:, :].astype(jnp.bfloat16)

        out_copies = {}
        emit_counter = [0]

        def emit(p, slot):
            i = emit_counter[0]
            emit_counter[0] += 1
            s = i % N_STAGE
            if s in out_copies:
                out_copies[s].wait()
            stage[s, :, :] = jnp.dot(
                bufs[p, slot], w_bf[:, :], preferred_element_type=jnp.float32
            )
            origin = me ^ g_tables[p][slot]
            cp = pltpu.make_async_copy(
                stage.at[s],
                out_hbm.at[pl.ds(origin * M_PER + p * ROWS, ROWS), :],
                out_sems.at[s],
            )
            cp.start()
            out_copies[s] = cp

        for p in range(N_PART):
            emit(p, 0)

        for ph in range(3):
            for p in PH_ORDERS[ph]:
                exchange_wait_recv(p, ph)
                if ph + 1 < 3:
                    sends.append(exchange_start(p, ph + 1))
                for slot in range(1 << ph, 2 << ph):
                    emit(p, slot)

        for rdma in sends:
            rdma.wait_send()
        for cp in out_copies.values():
            cp.wait()

    out_shape = jax.ShapeDtypeStruct((N_DEV * M_PER, n_per), jnp.float32)
    return pl.pallas_call(
        body,
        out_shape=out_shape,
        in_specs=[
            pl.BlockSpec(memory_space=pltpu.VMEM),
            pl.BlockSpec(memory_space=pl.ANY),
        ],
        out_specs=pl.BlockSpec(memory_space=pl.ANY),
        scratch_shapes=[
            pltpu.VMEM((N_PART, N_DEV, ROWS, k), jnp.bfloat16),
            pltpu.VMEM((k, 1024), jnp.float32),
            pltpu.VMEM((k, 1024), jnp.bfloat16),
            pltpu.VMEM((N_STAGE, ROWS, 1024), jnp.float32),
            pltpu.SemaphoreType.DMA((N_PART, 3)),
            pltpu.SemaphoreType.DMA((N_PART, 3)),
            pltpu.SemaphoreType.DMA,
            pltpu.SemaphoreType.DMA((N_STAGE,)),
        ],
        compiler_params=pltpu.CompilerParams(
            collective_id=0,
            vmem_limit_bytes=64 * 1024 * 1024,
        ),
    )(x, w_mat)


# device time: 153663 ns/iter; 1.2587x vs baseline; 1.0376x over previous
import jax
import jax.numpy as jnp
from jax import lax
from jax.experimental import pallas as pl
from jax.experimental.pallas import tpu as pltpu

try:
    jax.config.update("jax_compilation_cache_dir", "/tmp/scband_jax_cache")
    jax.config.update("jax_persistent_cache_min_compile_time_secs", 0.0)
except Exception:
    pass

N_DEV = 8
M_PER = 512
N_PART = 8
ROWS = M_PER // N_PART
N_STAGE = 4

X_MASK, Y_MASK, Z_MASK = 1, 3, 4

ORDERS = (
    (Z_MASK, Y_MASK, X_MASK),
    (Y_MASK, Z_MASK, X_MASK),
    (X_MASK, Z_MASK, Y_MASK),
    (X_MASK, Y_MASK, Z_MASK),
    (Y_MASK, X_MASK, Z_MASK),
    (Z_MASK, X_MASK, Y_MASK),
    (Z_MASK, Y_MASK, X_MASK),
    (X_MASK, Y_MASK, Z_MASK),
)

PH1_ORDER = (0, 1, 2, 3, 4, 5, 6, 7)
PH2_ORDER = (0, 3, 4, 1, 2, 5, 6, 7)
PH3_ORDER = (0, 3, 2, 1, 4, 5, 6, 7)
PH_ORDERS = (PH1_ORDER, PH2_ORDER, PH3_ORDER)


def _g_table(masks):
    m1, m2, m3 = masks
    g = [0, m1, m2, m2 ^ m1]
    return g + [m3 ^ v for v in g]


def kernel(x, w_mat):
    m_per, k = x.shape
    _, n_per = w_mat.shape
    assert m_per == M_PER and k == 4096 and n_per == 1024

    x = x.astype(jnp.bfloat16)

    def body(x_ref, w_hbm, out_hbm, bufs, w_f32, w_bf, stage,
             send_sems, recv_sems, w_sem, out_sems):
        me = lax.axis_index("i")

        w_dma = pltpu.make_async_copy(w_hbm, w_f32, w_sem)
        w_dma.start()

        for p in range(N_PART):
            bufs[p, 0, :, :] = x_ref[pl.ds(p * ROWS, ROWS), :]

        barrier_sem = pltpu.get_barrier_semaphore()
        for mask in (X_MASK, Y_MASK, Z_MASK):
            pl.semaphore_signal(
                barrier_sem, inc=1,
                device_id=(me ^ mask,), device_id_type=pl.DeviceIdType.MESH,
            )
        pl.semaphore_wait(barrier_sem, 3)

        def exchange_start(p, ph):
            cnt = 1 << ph
            nbr = me ^ ORDERS[p][ph]
            rdma = pltpu.make_async_remote_copy(
                src_ref=bufs.at[p, pl.ds(0, cnt)],
                dst_ref=bufs.at[p, pl.ds(cnt, cnt)],
                send_sem=send_sems.at[p, ph],
                recv_sem=recv_sems.at[p, ph],
                device_id=(nbr,), device_id_type=pl.DeviceIdType.MESH,
            )
            rdma.start()
            return rdma

        def exchange_wait_recv(p, ph):
            cnt = 1 << ph
            nbr = me ^ ORDERS[p][ph]
            pltpu.make_async_remote_copy(
                src_ref=bufs.at[p, pl.ds(0, cnt)],
                dst_ref=bufs.at[p, pl.ds(cnt, cnt)],
                send_sem=send_sems.at[p, ph],
                recv_sem=recv_sems.at[p, ph],
                device_id=(nbr,), device_id_type=pl.DeviceIdType.MESH,
            ).wait_recv()

        g_tables = [_g_table(ORDERS[p]) for p in range(N_PART)]

        sends = [exchange_start(p, 0) for p in range(N_PART)]

        w_dma.wait()
        w_bf[:, :] = w_f32[:, :].astype(jnp.bfloat16)

        out_copies = {}
        emit_counter = [0]

        def emit(p, slot):
            i = emit_counter[0]
            emit_counter[0] += 1
            s = i % N_STAGE
            if s in out_copies:
                out_copies[s].wait()
            stage[s, :, :] = jnp.dot(
                bufs[p, slot], w_bf[:, :], preferred_element_type=jnp.float32
            )
            origin = me ^ g_tables[p][slot]
            cp = pltpu.make_async_copy(
                stage.at[s],
                out_hbm.at[pl.ds(origin * M_PER + p * ROWS, ROWS), :],
                out_sems.at[s],
            )
            cp.start()
            out_copies[s] = cp

        for p in range(N_PART):
            emit(p, 0)

        for ph in range(3):
            for p in PH_ORDERS[ph]:
                exchange_wait_recv(p, ph)
                if ph + 1 < 3:
                    sends.append(exchange_start(p, ph + 1))
                for slot in range(1 << ph, 2 << ph):
                    emit(p, slot)

        for rdma in sends:
            rdma.wait_send()
        for cp in out_copies.values():
            cp.wait()

    out_shape = jax.ShapeDtypeStruct((N_DEV * M_PER, n_per), jnp.float32)
    return pl.pallas_call(
        body,
        out_shape=out_shape,
        in_specs=[
            pl.BlockSpec(memory_space=pltpu.VMEM),
            pl.BlockSpec(memory_space=pl.ANY),
        ],
        out_specs=pl.BlockSpec(memory_space=pl.ANY),
        scratch_shapes=[
            pltpu.VMEM((N_PART, N_DEV, ROWS, k), jnp.bfloat16),
            pltpu.VMEM((k, 1024), jnp.float32),
            pltpu.VMEM((k, 1024), jnp.bfloat16),
            pltpu.VMEM((N_STAGE, ROWS, 1024), jnp.float32),
            pltpu.SemaphoreType.DMA((N_PART, 3)),
            pltpu.SemaphoreType.DMA((N_PART, 3)),
            pltpu.SemaphoreType.DMA,
            pltpu.SemaphoreType.DMA((N_STAGE,)),
        ],
        compiler_params=pltpu.CompilerParams(
            collective_id=0,
            vmem_limit_bytes=64 * 1024 * 1024,
        ),
    )(x, w_mat)
